# Initial kernel scaffold; baseline (speedup 1.0000x reference)
import jax
import jax.numpy as jnp
from jax.experimental import pallas as pl


def kernel(x, edge_index, batch, W1, as1, ad1, b1, g1, be1, W2, as2, ad2, b2, g2, be2, W3, as3, ad3, b3, g3, be3, lnW, lnb, l0W, l0b, l1W, l1b):
    raise NotImplementedError("write your pallas kernel here")


# speedup vs baseline: 20.3494x; 20.3494x over previous
import functools
import jax
import jax.numpy as jnp
import numpy as np
from jax import lax
from jax.experimental import pallas as pl
from jax.experimental.pallas import tpu as pltpu
from jax.experimental.pallas import tpu_sc as plsc

N = 10000
D = 128
H = 8
C = 128
G = 64
HC = H * C
AW = 16

NC = 2
NS = 16
NW = NC * NS
L = 16
C2 = C // NC
HC2 = H * C2

R = 1000
NB = N // R

E_RAW = 160000
E_TOT = E_RAW + N
KPW = 5376
E_PAD = KPW * NW
B1 = 128
NBLK1 = KPW // B1
B2 = 64
K2 = E_PAD // NS
NBLK2 = K2 // B2
CH = 632
CHL = N - (NS - 1) * CH

PERM = np.array([h * C + half * C2 + j
                 for half in range(NC) for h in range(H) for j in range(C2)],
                dtype=np.int32)


def _prep_body(x_ref, w_ref, as_ref, ad_ref, xh0_ref, xh1_ref,
               als_ref, ald_ref):
    xh = jnp.dot(x_ref[...], w_ref[...], preferred_element_type=jnp.float32)
    xh0_ref[...] = xh[:, :HC2]
    xh1_ref[...] = xh[:, HC2:]
    als_ref[...] = jnp.dot(xh, as_ref[...], preferred_element_type=jnp.float32)
    ald_ref[...] = jnp.dot(xh, ad_ref[...], preferred_element_type=jnp.float32)


def _tc_prep(x, W, As, Ad):
    din = x.shape[1]
    return pl.pallas_call(
        _prep_body,
        grid=(NB,),
        in_specs=[
            pl.BlockSpec((R, din), lambda i: (i, 0)),
            pl.BlockSpec((din, HC), lambda i: (0, 0)),
            pl.BlockSpec((HC, AW), lambda i: (0, 0)),
            pl.BlockSpec((HC, AW), lambda i: (0, 0)),
        ],
        out_specs=[
            pl.BlockSpec((R, HC2), lambda i: (i, 0)),
            pl.BlockSpec((R, HC2), lambda i: (i, 0)),
            pl.BlockSpec((R, AW), lambda i: (i, 0)),
            pl.BlockSpec((R, AW), lambda i: (i, 0)),
        ],
        out_shape=[
            jax.ShapeDtypeStruct((N, HC2), jnp.float32),
            jax.ShapeDtypeStruct((N, HC2), jnp.float32),
            jax.ShapeDtypeStruct((N, AW), jnp.float32),
            jax.ShapeDtypeStruct((N, AW), jnp.float32),
        ],
    )(x, W, As, Ad)


def _mid_body(p0_ref, p1_ref, b_ref, g_ref, be_ref, w_ref, as_ref, ad_ref,
              xh0_ref, xh1_ref, als_ref, ald_ref, sum_ref, ssq_ref):
    ph = pl.program_id(0)
    i = pl.program_id(1)

    @pl.when(jnp.logical_and(ph == 0, i == 0))
    def _():
        sum_ref[...] = jnp.zeros_like(sum_ref)
        ssq_ref[...] = jnp.zeros_like(ssq_ref)

    t = jnp.maximum(
        jnp.concatenate([p0_ref[...], p1_ref[...]], axis=1) + b_ref[...], 0.0)

    @pl.when(ph == 0)
    def _():
        sum_ref[...] += jnp.sum(t, 0, keepdims=True)
        ssq_ref[...] += jnp.sum(t * t, 0, keepdims=True)

    @pl.when(ph == 1)
    def _():
        mu = sum_ref[...] * (1.0 / N)
        var = ssq_ref[...] * (1.0 / N) - mu * mu
        hn = (t - mu) * lax.rsqrt(var + 1e-5) * g_ref[...] + be_ref[...]
        xh = jnp.dot(hn, w_ref[...], preferred_element_type=jnp.float32)
        xh0_ref[...] = xh[:, :HC2]
        xh1_ref[...] = xh[:, HC2:]
        als_ref[...] = jnp.dot(xh, as_ref[...],
                               preferred_element_type=jnp.float32)
        ald_ref[...] = jnp.dot(xh, ad_ref[...],
                               preferred_element_type=jnp.float32)


def _tc_mid(p0, p1, b, g, be, W, As, Ad):
    return pl.pallas_call(
        _mid_body,
        grid=(2, NB),
        in_specs=[
            pl.BlockSpec((R, C2), lambda p, i: (i, 0)),
            pl.BlockSpec((R, C2), lambda p, i: (i, 0)),
            pl.BlockSpec((1, C), lambda p, i: (0, 0)),
            pl.BlockSpec((1, C), lambda p, i: (0, 0)),
            pl.BlockSpec((1, C), lambda p, i: (0, 0)),
            pl.BlockSpec((C, HC), lambda p, i: (0, 0)),
            pl.BlockSpec((HC, AW), lambda p, i: (0, 0)),
            pl.BlockSpec((HC, AW), lambda p, i: (0, 0)),
        ],
        out_specs=[
            pl.BlockSpec((R, HC2), lambda p, i: (i, 0)),
            pl.BlockSpec((R, HC2), lambda p, i: (i, 0)),
            pl.BlockSpec((R, AW), lambda p, i: (i, 0)),
            pl.BlockSpec((R, AW), lambda p, i: (i, 0)),
        ],
        out_shape=[
            jax.ShapeDtypeStruct((N, HC2), jnp.float32),
            jax.ShapeDtypeStruct((N, HC2), jnp.float32),
            jax.ShapeDtypeStruct((N, AW), jnp.float32),
            jax.ShapeDtypeStruct((N, AW), jnp.float32),
        ],
        scratch_shapes=[
            pltpu.VMEM((1, C), jnp.float32),
            pltpu.VMEM((1, C), jnp.float32),
        ],
        compiler_params=pltpu.CompilerParams(
            dimension_semantics=("arbitrary", "arbitrary")),
    )(p0, p1, b, g, be, W, As, Ad)


def _sc_pass1(als, ald, srcp, dstp, z16):
    mesh = plsc.VectorSubcoreMesh(core_axis_name="c", subcore_axis_name="s")

    @functools.partial(
        pl.kernel,
        out_type=(
            jax.ShapeDtypeStruct((E_PAD, AW), jnp.float32),
            jax.ShapeDtypeStruct((NC, N, AW), jnp.float32),
        ),
        mesh=mesh,
        scratch_types=[
            pltpu.VMEM((B1,), jnp.int32),
            pltpu.VMEM((B1,), jnp.int32),
            pltpu.VMEM((B1, AW), jnp.float32),
            pltpu.VMEM((B1, AW), jnp.float32),
            pltpu.VMEM((B1, AW), jnp.float32),
            pltpu.VMEM_SHARED((N, AW), jnp.float32),
            pltpu.SemaphoreType.DMA,
            pltpu.SemaphoreType.DMA,
        ],
        compiler_params=pltpu.CompilerParams(use_tc_tiling_on_sc=False),
    )
    def k(als_h, ald_h, src_h, dst_h, z_h, ex_h, den_h,
          idx_s, idx_d, sbuf, dbuf, exbuf, den_sh, sem1, sem2):
        c = lax.axis_index("c")
        s = lax.axis_index("s")
        wid = s * NC + c
        offs = pl.multiple_of(s * CH, 8)

        @pl.when(s < NS - 1)
        def _():
            pltpu.sync_copy(z_h.at[pl.ds(offs, CH), :],
                            den_sh.at[pl.ds(offs, CH), :])

        @pl.when(s == NS - 1)
        def _():
            pltpu.sync_copy(z_h.at[pl.ds(offs, CHL), :],
                            den_sh.at[pl.ds(offs, CHL), :])

        plsc.subcore_barrier()
        base_w = wid * KPW

        def blk(i, carry):
            base = base_w + i * B1
            pltpu.sync_copy(src_h.at[pl.ds(base, B1)], idx_s)
            pltpu.sync_copy(dst_h.at[pl.ds(base, B1)], idx_d)
            cp1 = pltpu.async_copy(als_h.at[idx_s], sbuf, sem1)
            cp2 = pltpu.async_copy(ald_h.at[idx_d], dbuf, sem2)
            cp1.wait()
            cp2.wait()
            for j in range(B1):
                a = sbuf[j, :] + dbuf[j, :]
                a = jnp.maximum(a, 0.2 * a)
                ev = jnp.exp(a)
                ev = jnp.where(base + j < E_TOT, ev, 0.0)
                exbuf[j, :] = ev
            pltpu.sync_copy(exbuf, den_sh.at[idx_d], add=True)
            pltpu.sync_copy(exbuf, ex_h.at[pl.ds(base, B1), :])
            return carry

        lax.fori_loop(0, NBLK1, blk, 0)
        plsc.subcore_barrier()

        @pl.when(s < NS - 1)
        def _():
            pltpu.sync_copy(den_sh.at[pl.ds(offs, CH), :],
                            den_h.at[c, pl.ds(offs, CH), :])

        @pl.when(s == NS - 1)
        def _():
            pltpu.sync_copy(den_sh.at[pl.ds(offs, CHL), :],
                            den_h.at[c, pl.ds(offs, CHL), :])

    return k(als, ald, srcp, dstp, z16)


def _sc_pass2(xh0, xh1, ex, d0, d1, srcp, dstp, z64):
    mesh = plsc.VectorSubcoreMesh(core_axis_name="c", subcore_axis_name="s")

    @functools.partial(
        pl.kernel,
        out_type=(
            jax.ShapeDtypeStruct((N, C2), jnp.float32),
            jax.ShapeDtypeStruct((N, C2), jnp.float32),
        ),
        mesh=mesh,
        scratch_types=[
            pltpu.VMEM((B2,), jnp.int32),
            pltpu.VMEM((B2,), jnp.int32),
            pltpu.VMEM((B2, HC2), jnp.float32),
            pltpu.VMEM((B2, AW), jnp.float32),
            pltpu.VMEM((B2, AW), jnp.float32),
            pltpu.VMEM((B2, AW), jnp.float32),
            pltpu.VMEM((B2, AW), jnp.float32),
            pltpu.VMEM((B2, C2), jnp.float32),
            pltpu.VMEM_SHARED((N, C2), jnp.float32),
            pltpu.SemaphoreType.DMA,
            pltpu.SemaphoreType.DMA,
            pltpu.SemaphoreType.DMA,
        ],
        compiler_params=pltpu.CompilerParams(use_tc_tiling_on_sc=False),
    )
    def k(xh0_h, xh1_h, ex_h, d0_h, d1_h, src_h, dst_h, z_h,
          out0_h, out1_h,
          idx_s, idx_d, rows, exb, d0b, d1b, wb, msg, out_sh,
          sem_r, sem_a, sem_b):
        c = lax.axis_index("c")
        s = lax.axis_index("s")
        offs = pl.multiple_of(s * CH, 8)

        @pl.when(s < NS - 1)
        def _():
            pltpu.sync_copy(z_h.at[pl.ds(offs, CH), :],
                            out_sh.at[pl.ds(offs, CH), :])

        @pl.when(s == NS - 1)
        def _():
            pltpu.sync_copy(z_h.at[pl.ds(offs, CHL), :],
                            out_sh.at[pl.ds(offs, CHL), :])

        plsc.subcore_barrier()
        base_w = s * K2

        def make_blk(xh_h):
            def blk(i, carry):
                base = base_w + i * B2
                pltpu.sync_copy(src_h.at[pl.ds(base, B2)], idx_s)
                pltpu.sync_copy(dst_h.at[pl.ds(base, B2)], idx_d)
                cpr = pltpu.async_copy(xh_h.at[idx_s], rows, sem_r)
                cpa = pltpu.async_copy(d0_h.at[idx_d], d0b, sem_a)
                cpb = pltpu.async_copy(d1_h.at[idx_d], d1b, sem_b)
                pltpu.sync_copy(ex_h.at[pl.ds(base, B2), :], exb)
                cpa.wait()
                cpb.wait()
                for j in range(B2):
                    w = exb[j, :] * (1.0 / H) / (
                        d0b[j, :] + d1b[j, :] + 1e-16)
                    wb[j, :] = w
                cpr.wait()

                def edge(j, carry2):
                    wv = wb[j, :]
                    acc = [jnp.zeros((L,), jnp.float32)
                           for _ in range(C2 // L)]
                    for h in range(H):
                        sv = jnp.full((L,), wv[h])
                        for kk in range(C2 // L):
                            acc[kk] = acc[kk] + sv * rows[
                                j, pl.ds(h * C2 + kk * L, L)]
                    for kk in range(C2 // L):
                        msg[j, pl.ds(kk * L, L)] = acc[kk]
                    return carry2

                lax.fori_loop(0, B2, edge, 0)
                pltpu.sync_copy(msg, out_sh.at[idx_d], add=True)
                return carry
            return blk

        @pl.when(c == 0)
        def _():
            lax.fori_loop(0, NBLK2, make_blk(xh0_h), 0)

        @pl.when(c == 1)
        def _():
            lax.fori_loop(0, NBLK2, make_blk(xh1_h), 0)

        plsc.subcore_barrier()

        @pl.when(c == 0)
        def _():
            @pl.when(s < NS - 1)
            def _():
                pltpu.sync_copy(out_sh.at[pl.ds(offs, CH), :],
                                out0_h.at[pl.ds(offs, CH), :])

            @pl.when(s == NS - 1)
            def _():
                pltpu.sync_copy(out_sh.at[pl.ds(offs, CHL), :],
                                out0_h.at[pl.ds(offs, CHL), :])

        @pl.when(c == 1)
        def _():
            @pl.when(s < NS - 1)
            def _():
                pltpu.sync_copy(out_sh.at[pl.ds(offs, CH), :],
                                out1_h.at[pl.ds(offs, CH), :])

            @pl.when(s == NS - 1)
            def _():
                pltpu.sync_copy(out_sh.at[pl.ds(offs, CHL), :],
                                out1_h.at[pl.ds(offs, CHL), :])

    return k(xh0, xh1, ex, d0, d1, srcp, dstp, z64)


def _fin_body(p0_ref, p1_ref, b_ref, g_ref, be_ref, bat_ref, x_ref,
              lnw_ref, lnb_ref, l0w_ref, l0b_ref, l1w_ref, l1b_ref,
              out_ref,
              sum_ref, ssq_ref, gmax_ref, gsum_ref, cnt_ref, root_ref,
              xr_ref):
    ph = pl.program_id(0)
    i = pl.program_id(1)

    @pl.when(jnp.logical_and(ph == 0, i == 0))
    def _():
        sum_ref[...] = jnp.zeros_like(sum_ref)
        ssq_ref[...] = jnp.zeros_like(ssq_ref)
        gmax_ref[...] = jnp.full_like(gmax_ref, -jnp.inf)
        gsum_ref[...] = jnp.zeros_like(gsum_ref)
        cnt_ref[...] = jnp.zeros_like(cnt_ref)
        root_ref[...] = jnp.full_like(root_ref, jnp.inf)

    t = jnp.maximum(
        jnp.concatenate([p0_ref[...], p1_ref[...]], axis=1) + b_ref[...], 0.0)

    @pl.when(ph == 0)
    def _():
        sum_ref[...] += jnp.sum(t, 0, keepdims=True)
        ssq_ref[...] += jnp.sum(t * t, 0, keepdims=True)

    @pl.when(ph == 1)
    def _():
        mu = sum_ref[...] * (1.0 / N)
        var = ssq_ref[...] * (1.0 / N) - mu * mu
        h = (t - mu) * lax.rsqrt(var + 1e-5) * g_ref[...] + be_ref[...]
        bat = bat_ref[...]
        rowf = (i * R + lax.broadcasted_iota(jnp.int32, (R, 1), 0)
                ).astype(jnp.float32)
        bmin = jnp.min(bat).astype(jnp.int32)
        bmax = jnp.max(bat).astype(jnp.int32)

        def body(gi, carry):
            @pl.when(jnp.logical_and(gi >= bmin, gi <= bmax))
            def _():
                mask = bat == gi.astype(jnp.float32)
                hm = jnp.where(mask, h, -jnp.inf)
                gmax_ref[pl.ds(gi, 1), :] = jnp.maximum(
                    gmax_ref[pl.ds(gi, 1), :], jnp.max(hm, 0, keepdims=True))
                hs = jnp.where(mask, h, 0.0)
                gsum_ref[pl.ds(gi, 1), :] += jnp.sum(hs, 0, keepdims=True)
                cnt_ref[pl.ds(gi, 1), :] += jnp.sum(
                    mask.astype(jnp.float32), 0, keepdims=True)
                ridx = jnp.where(mask, rowf, jnp.inf)
                root_ref[pl.ds(gi, 1), :] = jnp.minimum(
                    root_ref[pl.ds(gi, 1), :], jnp.min(ridx, 0, keepdims=True))
            return carry

        lax.fori_loop(0, G, body, 0)

    @pl.when(jnp.logical_and(ph == 2, i == 0))
    def _():
        gmax = gmax_ref[...]
        gmax = jnp.where(jnp.isfinite(gmax), gmax, 0.0)
        gmean = gsum_ref[...] / jnp.maximum(cnt_ref[...], 1.0)
        hgin = jnp.concatenate([gmax, gmean], axis=1)
        hg = jnp.maximum(
            jnp.dot(hgin, l0w_ref[...], preferred_element_type=jnp.float32)
            + l0b_ref[...], 0.0)
        for gi in range(G):
            ri = jnp.minimum(root_ref[gi, 0], float(N - 1)).astype(jnp.int32)
            xr_ref[pl.ds(gi, 1), :] = x_ref[pl.ds(ri, 1), :]
        news = jnp.maximum(
            jnp.dot(xr_ref[...], lnw_ref[...],
                    preferred_element_type=jnp.float32) + lnb_ref[...], 0.0)
        z = jnp.concatenate([hg, news], axis=1)
        val = jnp.dot(z, l1w_ref[...],
                      preferred_element_type=jnp.float32) + l1b_ref[0]
        out_ref[...] = jax.nn.sigmoid(val)


def _tc_final(p0, p1, b, g, be, batf, x, lnW, lnb, l0W, l0b, l1W, l1b):
    return pl.pallas_call(
        _fin_body,
        grid=(3, NB),
        in_specs=[
            pl.BlockSpec((R, C2), lambda p, i: (i, 0)),
            pl.BlockSpec((R, C2), lambda p, i: (i, 0)),
            pl.BlockSpec((1, C), lambda p, i: (0, 0)),
            pl.BlockSpec((1, C), lambda p, i: (0, 0)),
            pl.BlockSpec((1, C), lambda p, i: (0, 0)),
            pl.BlockSpec((R, 1), lambda p, i: (i, 0)),
            pl.BlockSpec((N, D), lambda p, i: (0, 0)),
            pl.BlockSpec((D, C), lambda p, i: (0, 0)),
            pl.BlockSpec((1, C), lambda p, i: (0, 0)),
            pl.BlockSpec((2 * C, C), lambda p, i: (0, 0)),
            pl.BlockSpec((1, C), lambda p, i: (0, 0)),
            pl.BlockSpec((2 * C, 1), lambda p, i: (0, 0)),
            pl.BlockSpec(memory_space=pltpu.SMEM),
        ],
        out_specs=pl.BlockSpec((G, 1), lambda p, i: (0, 0)),
        out_shape=jax.ShapeDtypeStruct((G, 1), jnp.float32),
        scratch_shapes=[
            pltpu.VMEM((1, C), jnp.float32),
            pltpu.VMEM((1, C), jnp.float32),
            pltpu.VMEM((G, C), jnp.float32),
            pltpu.VMEM((G, C), jnp.float32),
            pltpu.VMEM((G, 1), jnp.float32),
            pltpu.VMEM((G, 1), jnp.float32),
            pltpu.VMEM((G, D), jnp.float32),
        ],
        compiler_params=pltpu.CompilerParams(
            dimension_semantics=("arbitrary", "arbitrary")),
    )(p0, p1, b, g, be, batf, x, lnW, lnb, l0W, l0b, l1W, l1b)


def _block_diag(a):
    m = a[:, :, None] * jnp.eye(H, dtype=a.dtype)[:, None, :]
    return jnp.pad(m.reshape(HC, H), ((0, 0), (0, AW - H)))


def _gat_layer(xh0, xh1, als, ald, srcp, dstp, z16, z64):
    ex, den = _sc_pass1(als, ald, srcp, dstp, z16)
    return _sc_pass2(xh0, xh1, ex, den[0], den[1], srcp, dstp, z64)


def kernel(x, edge_index, batch, W1, as1, ad1, b1, g1, be1,
           W2, as2, ad2, b2, g2, be2, W3, as3, ad3, b3, g3, be3,
           lnW, lnb, l0W, l0b, l1W, l1b):
    loop = jnp.arange(N, dtype=edge_index.dtype)
    pad = jnp.zeros((E_PAD - E_TOT,), edge_index.dtype)
    srcp = jnp.concatenate([edge_index[0], loop, pad])
    dstp = jnp.concatenate([edge_index[1], loop, pad])
    z16 = jnp.zeros((N, AW), jnp.float32)
    z64 = jnp.zeros((N, C2), jnp.float32)
    batf = batch.astype(jnp.float32).reshape(N, 1)
    perm = jnp.asarray(PERM)

    xh0, xh1, als, ald = _tc_prep(
        x, W1[:, perm], _block_diag(as1)[perm, :], _block_diag(ad1)[perm, :])
    p0, p1 = _gat_layer(xh0, xh1, als, ald, srcp, dstp, z16, z64)
    xh0, xh1, als, ald = _tc_mid(
        p0, p1, b1.reshape(1, C), g1.reshape(1, C), be1.reshape(1, C),
        W2[:, perm], _block_diag(as2)[perm, :], _block_diag(ad2)[perm, :])
    p0, p1 = _gat_layer(xh0, xh1, als, ald, srcp, dstp, z16, z64)
    xh0, xh1, als, ald = _tc_mid(
        p0, p1, b2.reshape(1, C), g2.reshape(1, C), be2.reshape(1, C),
        W3[:, perm], _block_diag(as3)[perm, :], _block_diag(ad3)[perm, :])
    p0, p1 = _gat_layer(xh0, xh1, als, ald, srcp, dstp, z16, z64)
    return _tc_final(p0, p1, b3.reshape(1, C), g3.reshape(1, C),
                     be3.reshape(1, C), batf, x,
                     lnW, lnb.reshape(1, C), l0W, l0b.reshape(1, C),
                     l1W, l1b)


# speedup vs baseline: 26.6032x; 1.3073x over previous
import functools
import jax
import jax.numpy as jnp
import numpy as np
from jax import lax
from jax.experimental import pallas as pl
from jax.experimental.pallas import tpu as pltpu
from jax.experimental.pallas import tpu_sc as plsc

N = 10000
D = 128
H = 8
C = 128
G = 64
HC = H * C
AW = 16

NC = 2
NS = 16
NW = NC * NS
L = 16
C2 = C // NC
HC2 = H * C2

R = 1000
NB = N // R

E_RAW = 160000
E_TOT = E_RAW + N
KPW = 5376
E_PAD = KPW * NW
B1 = 128
NBLK1 = KPW // B1
B2 = 64
K2 = E_PAD // NS
NBLK2 = K2 // B2
CH = 632
CHL = N - (NS - 1) * CH

PERM = np.array([h * C + half * C2 + j
                 for half in range(NC) for h in range(H) for j in range(C2)],
                dtype=np.int32)


def _prep_body(x_ref, w_ref, as_ref, ad_ref, xh0_ref, xh1_ref,
               als_ref, ald_ref):
    xh = jnp.dot(x_ref[...], w_ref[...], preferred_element_type=jnp.float32)
    xh0_ref[...] = xh[:, :HC2]
    xh1_ref[...] = xh[:, HC2:]
    als_ref[...] = jnp.dot(xh, as_ref[...], preferred_element_type=jnp.float32)
    ald_ref[...] = jnp.dot(xh, ad_ref[...], preferred_element_type=jnp.float32)


def _tc_prep(x, W, As, Ad):
    din = x.shape[1]
    return pl.pallas_call(
        _prep_body,
        grid=(NB,),
        in_specs=[
            pl.BlockSpec((R, din), lambda i: (i, 0)),
            pl.BlockSpec((din, HC), lambda i: (0, 0)),
            pl.BlockSpec((HC, AW), lambda i: (0, 0)),
            pl.BlockSpec((HC, AW), lambda i: (0, 0)),
        ],
        out_specs=[
            pl.BlockSpec((R, HC2), lambda i: (i, 0)),
            pl.BlockSpec((R, HC2), lambda i: (i, 0)),
            pl.BlockSpec((R, AW), lambda i: (i, 0)),
            pl.BlockSpec((R, AW), lambda i: (i, 0)),
        ],
        out_shape=[
            jax.ShapeDtypeStruct((N, HC2), jnp.float32),
            jax.ShapeDtypeStruct((N, HC2), jnp.float32),
            jax.ShapeDtypeStruct((N, AW), jnp.float32),
            jax.ShapeDtypeStruct((N, AW), jnp.float32),
        ],
    )(x, W, As, Ad)


def _mid_body(p0_ref, p1_ref, b_ref, g_ref, be_ref, w_ref, as_ref, ad_ref,
              xh0_ref, xh1_ref, als_ref, ald_ref, sum_ref, ssq_ref):
    ph = pl.program_id(0)
    i = pl.program_id(1)

    @pl.when(jnp.logical_and(ph == 0, i == 0))
    def _():
        sum_ref[...] = jnp.zeros_like(sum_ref)
        ssq_ref[...] = jnp.zeros_like(ssq_ref)

    t = jnp.maximum(
        jnp.concatenate([p0_ref[...], p1_ref[...]], axis=1) + b_ref[...], 0.0)

    @pl.when(ph == 0)
    def _():
        sum_ref[...] += jnp.sum(t, 0, keepdims=True)
        ssq_ref[...] += jnp.sum(t * t, 0, keepdims=True)

    @pl.when(ph == 1)
    def _():
        mu = sum_ref[...] * (1.0 / N)
        var = ssq_ref[...] * (1.0 / N) - mu * mu
        hn = (t - mu) * lax.rsqrt(var + 1e-5) * g_ref[...] + be_ref[...]
        xh = jnp.dot(hn, w_ref[...], preferred_element_type=jnp.float32)
        xh0_ref[...] = xh[:, :HC2]
        xh1_ref[...] = xh[:, HC2:]
        als_ref[...] = jnp.dot(xh, as_ref[...],
                               preferred_element_type=jnp.float32)
        ald_ref[...] = jnp.dot(xh, ad_ref[...],
                               preferred_element_type=jnp.float32)


def _tc_mid(p0, p1, b, g, be, W, As, Ad):
    return pl.pallas_call(
        _mid_body,
        grid=(2, NB),
        in_specs=[
            pl.BlockSpec((R, C2), lambda p, i: (i, 0)),
            pl.BlockSpec((R, C2), lambda p, i: (i, 0)),
            pl.BlockSpec((1, C), lambda p, i: (0, 0)),
            pl.BlockSpec((1, C), lambda p, i: (0, 0)),
            pl.BlockSpec((1, C), lambda p, i: (0, 0)),
            pl.BlockSpec((C, HC), lambda p, i: (0, 0)),
            pl.BlockSpec((HC, AW), lambda p, i: (0, 0)),
            pl.BlockSpec((HC, AW), lambda p, i: (0, 0)),
        ],
        out_specs=[
            pl.BlockSpec((R, HC2), lambda p, i: (i, 0)),
            pl.BlockSpec((R, HC2), lambda p, i: (i, 0)),
            pl.BlockSpec((R, AW), lambda p, i: (i, 0)),
            pl.BlockSpec((R, AW), lambda p, i: (i, 0)),
        ],
        out_shape=[
            jax.ShapeDtypeStruct((N, HC2), jnp.float32),
            jax.ShapeDtypeStruct((N, HC2), jnp.float32),
            jax.ShapeDtypeStruct((N, AW), jnp.float32),
            jax.ShapeDtypeStruct((N, AW), jnp.float32),
        ],
        scratch_shapes=[
            pltpu.VMEM((1, C), jnp.float32),
            pltpu.VMEM((1, C), jnp.float32),
        ],
        compiler_params=pltpu.CompilerParams(
            dimension_semantics=("arbitrary", "arbitrary")),
    )(p0, p1, b, g, be, W, As, Ad)


def _sc_pass1(als, ald, srcp, dstp, z16):
    mesh = plsc.VectorSubcoreMesh(core_axis_name="c", subcore_axis_name="s")

    @functools.partial(
        pl.kernel,
        out_type=(
            jax.ShapeDtypeStruct((E_PAD, AW), jnp.float32),
            jax.ShapeDtypeStruct((NC, N, AW), jnp.float32),
        ),
        mesh=mesh,
        scratch_types=[
            pltpu.VMEM((B1,), jnp.int32),
            pltpu.VMEM((B1,), jnp.int32),
            pltpu.VMEM((B1, AW), jnp.float32),
            pltpu.VMEM((B1, AW), jnp.float32),
            pltpu.VMEM((B1, AW), jnp.float32),
            pltpu.VMEM_SHARED((N, AW), jnp.float32),
            pltpu.SemaphoreType.DMA,
            pltpu.SemaphoreType.DMA,
        ],
        compiler_params=pltpu.CompilerParams(use_tc_tiling_on_sc=False),
    )
    def k(als_h, ald_h, src_h, dst_h, z_h, ex_h, den_h,
          idx_s, idx_d, sbuf, dbuf, exbuf, den_sh, sem1, sem2):
        c = lax.axis_index("c")
        s = lax.axis_index("s")
        wid = s * NC + c
        offs = pl.multiple_of(s * CH, 8)

        @pl.when(s < NS - 1)
        def _():
            pltpu.sync_copy(z_h.at[pl.ds(offs, CH), :],
                            den_sh.at[pl.ds(offs, CH), :])

        @pl.when(s == NS - 1)
        def _():
            pltpu.sync_copy(z_h.at[pl.ds(offs, CHL), :],
                            den_sh.at[pl.ds(offs, CHL), :])

        plsc.subcore_barrier()
        base_w = wid * KPW

        def blk(i, carry):
            base = base_w + i * B1
            pltpu.sync_copy(src_h.at[pl.ds(base, B1)], idx_s)
            pltpu.sync_copy(dst_h.at[pl.ds(base, B1)], idx_d)
            cp1 = pltpu.async_copy(als_h.at[idx_s], sbuf, sem1)
            cp2 = pltpu.async_copy(ald_h.at[idx_d], dbuf, sem2)
            cp1.wait()
            cp2.wait()
            for j in range(B1):
                a = sbuf[j, :] + dbuf[j, :]
                a = jnp.maximum(a, 0.2 * a)
                ev = jnp.exp(a)
                ev = jnp.where(base + j < E_TOT, ev, 0.0)
                exbuf[j, :] = ev
            pltpu.sync_copy(exbuf, den_sh.at[idx_d], add=True)
            pltpu.sync_copy(exbuf, ex_h.at[pl.ds(base, B1), :])
            return carry

        lax.fori_loop(0, NBLK1, blk, 0)
        plsc.subcore_barrier()

        @pl.when(s < NS - 1)
        def _():
            pltpu.sync_copy(den_sh.at[pl.ds(offs, CH), :],
                            den_h.at[c, pl.ds(offs, CH), :])

        @pl.when(s == NS - 1)
        def _():
            pltpu.sync_copy(den_sh.at[pl.ds(offs, CHL), :],
                            den_h.at[c, pl.ds(offs, CHL), :])

    return k(als, ald, srcp, dstp, z16)


def _sc_pass2(xh0, xh1, ex, d0, d1, srcp, dstp, z64):
    mesh = plsc.VectorSubcoreMesh(core_axis_name="c", subcore_axis_name="s")

    @functools.partial(
        pl.kernel,
        out_type=(
            jax.ShapeDtypeStruct((N, C2), jnp.float32),
            jax.ShapeDtypeStruct((N, C2), jnp.float32),
        ),
        mesh=mesh,
        scratch_types=(
            [pltpu.VMEM((B2,), jnp.int32)] * 4
            + [pltpu.VMEM((B2, HC2), jnp.float32)] * 2
            + [pltpu.VMEM((B2, AW), jnp.float32)] * 8
            + [
                pltpu.VMEM((B2, C2), jnp.float32),
                pltpu.VMEM_SHARED((N, C2), jnp.float32),
            ]
            + [pltpu.SemaphoreType.DMA] * 6
        ),
        compiler_params=pltpu.CompilerParams(use_tc_tiling_on_sc=False),
    )
    def k(xh0_h, xh1_h, ex_h, d0_h, d1_h, src_h, dst_h, z_h,
          out0_h, out1_h,
          idx_s0, idx_s1, idx_d0, idx_d1, rows0, rows1,
          exb0, exb1, d0b0, d0b1, d1b0, d1b1, wb0, wb1, msg, out_sh,
          sem_r0, sem_r1, sem_a0, sem_a1, sem_b0, sem_b1):
        c = lax.axis_index("c")
        s = lax.axis_index("s")
        offs = pl.multiple_of(s * CH, 8)

        @pl.when(s < NS - 1)
        def _():
            pltpu.sync_copy(z_h.at[pl.ds(offs, CH), :],
                            out_sh.at[pl.ds(offs, CH), :])

        @pl.when(s == NS - 1)
        def _():
            pltpu.sync_copy(z_h.at[pl.ds(offs, CHL), :],
                            out_sh.at[pl.ds(offs, CHL), :])

        plsc.subcore_barrier()
        base_w = s * K2
        bufs = (
            (idx_s0, idx_d0, rows0, exb0, d0b0, d1b0, wb0,
             sem_r0, sem_a0, sem_b0),
            (idx_s1, idx_d1, rows1, exb1, d0b1, d1b1, wb1,
             sem_r1, sem_a1, sem_b1),
        )

        def run_core(xh_h):
            def fire(i, b):
                idx_s, idx_d, rows, exb, d0b, d1b, wb, sem_r, sem_a, sem_b \
                    = bufs[b]
                base = base_w + i * B2
                pltpu.sync_copy(src_h.at[pl.ds(base, B2)], idx_s)
                pltpu.sync_copy(dst_h.at[pl.ds(base, B2)], idx_d)
                pltpu.async_copy(xh_h.at[idx_s], rows, sem_r)
                pltpu.async_copy(d0_h.at[idx_d], d0b, sem_a)
                pltpu.async_copy(d1_h.at[idx_d], d1b, sem_b)
                pltpu.sync_copy(ex_h.at[pl.ds(base, B2), :], exb)

            def consume(b):
                idx_s, idx_d, rows, exb, d0b, d1b, wb, sem_r, sem_a, sem_b \
                    = bufs[b]
                pltpu.make_async_copy(d0_h.at[idx_d], d0b, sem_a).wait()
                pltpu.make_async_copy(d1_h.at[idx_d], d1b, sem_b).wait()
                for j in range(B2):
                    wb[j, :] = exb[j, :] * (1.0 / H) / (
                        d0b[j, :] + d1b[j, :] + 1e-16)
                pltpu.make_async_copy(xh_h.at[idx_s], rows, sem_r).wait()

                @plsc.parallel_loop(0, B2, unroll=2)
                def _edge(j):
                    wv = wb[j, :]
                    acc = [jnp.zeros((L,), jnp.float32)
                           for _ in range(C2 // L)]
                    for h in range(H):
                        sv = jnp.full((L,), wv[h])
                        for kk in range(C2 // L):
                            acc[kk] = acc[kk] + sv * rows[
                                j, pl.ds(h * C2 + kk * L, L)]
                    for kk in range(C2 // L):
                        msg[j, pl.ds(kk * L, L)] = acc[kk]

                pltpu.sync_copy(msg, out_sh.at[idx_d], add=True)

            fire(0, 0)

            def superblk(si, carry):
                for b in range(2):
                    i = 2 * si + b

                    @pl.when(i + 1 < NBLK2)
                    def _():
                        fire(i + 1, 1 - b)

                    consume(b)
                return carry

            lax.fori_loop(0, NBLK2 // 2, superblk, 0)

        @pl.when(c == 0)
        def _():
            run_core(xh0_h)

        @pl.when(c == 1)
        def _():
            run_core(xh1_h)

        plsc.subcore_barrier()

        @pl.when(c == 0)
        def _():
            @pl.when(s < NS - 1)
            def _():
                pltpu.sync_copy(out_sh.at[pl.ds(offs, CH), :],
                                out0_h.at[pl.ds(offs, CH), :])

            @pl.when(s == NS - 1)
            def _():
                pltpu.sync_copy(out_sh.at[pl.ds(offs, CHL), :],
                                out0_h.at[pl.ds(offs, CHL), :])

        @pl.when(c == 1)
        def _():
            @pl.when(s < NS - 1)
            def _():
                pltpu.sync_copy(out_sh.at[pl.ds(offs, CH), :],
                                out1_h.at[pl.ds(offs, CH), :])

            @pl.when(s == NS - 1)
            def _():
                pltpu.sync_copy(out_sh.at[pl.ds(offs, CHL), :],
                                out1_h.at[pl.ds(offs, CHL), :])

    return k(xh0, xh1, ex, d0, d1, srcp, dstp, z64)


def _fin_body(p0_ref, p1_ref, b_ref, g_ref, be_ref, bat_ref, x_ref,
              lnw_ref, lnb_ref, l0w_ref, l0b_ref, l1w_ref, l1b_ref,
              out_ref,
              sum_ref, ssq_ref, gmax_ref, gsum_ref, cnt_ref, root_ref,
              xr_ref):
    ph = pl.program_id(0)
    i = pl.program_id(1)

    @pl.when(jnp.logical_and(ph == 0, i == 0))
    def _():
        sum_ref[...] = jnp.zeros_like(sum_ref)
        ssq_ref[...] = jnp.zeros_like(ssq_ref)
        gmax_ref[...] = jnp.full_like(gmax_ref, -jnp.inf)
        gsum_ref[...] = jnp.zeros_like(gsum_ref)
        cnt_ref[...] = jnp.zeros_like(cnt_ref)
        root_ref[...] = jnp.full_like(root_ref, jnp.inf)

    t = jnp.maximum(
        jnp.concatenate([p0_ref[...], p1_ref[...]], axis=1) + b_ref[...], 0.0)

    @pl.when(ph == 0)
    def _():
        sum_ref[...] += jnp.sum(t, 0, keepdims=True)
        ssq_ref[...] += jnp.sum(t * t, 0, keepdims=True)

    @pl.when(ph == 1)
    def _():
        mu = sum_ref[...] * (1.0 / N)
        var = ssq_ref[...] * (1.0 / N) - mu * mu
        h = (t - mu) * lax.rsqrt(var + 1e-5) * g_ref[...] + be_ref[...]
        bat = bat_ref[...]
        rowf = (i * R + lax.broadcasted_iota(jnp.int32, (R, 1), 0)
                ).astype(jnp.float32)
        bmin = jnp.min(bat).astype(jnp.int32)
        bmax = jnp.max(bat).astype(jnp.int32)

        def body(gi, carry):
            @pl.when(jnp.logical_and(gi >= bmin, gi <= bmax))
            def _():
                mask = bat == gi.astype(jnp.float32)
                hm = jnp.where(mask, h, -jnp.inf)
                gmax_ref[pl.ds(gi, 1), :] = jnp.maximum(
                    gmax_ref[pl.ds(gi, 1), :], jnp.max(hm, 0, keepdims=True))
                hs = jnp.where(mask, h, 0.0)
                gsum_ref[pl.ds(gi, 1), :] += jnp.sum(hs, 0, keepdims=True)
                cnt_ref[pl.ds(gi, 1), :] += jnp.sum(
                    mask.astype(jnp.float32), 0, keepdims=True)
                ridx = jnp.where(mask, rowf, jnp.inf)
                root_ref[pl.ds(gi, 1), :] = jnp.minimum(
                    root_ref[pl.ds(gi, 1), :], jnp.min(ridx, 0, keepdims=True))
            return carry

        lax.fori_loop(0, G, body, 0)

    @pl.when(jnp.logical_and(ph == 2, i == 0))
    def _():
        gmax = gmax_ref[...]
        gmax = jnp.where(jnp.isfinite(gmax), gmax, 0.0)
        gmean = gsum_ref[...] / jnp.maximum(cnt_ref[...], 1.0)
        hgin = jnp.concatenate([gmax, gmean], axis=1)
        hg = jnp.maximum(
            jnp.dot(hgin, l0w_ref[...], preferred_element_type=jnp.float32)
            + l0b_ref[...], 0.0)
        for gi in range(G):
            ri = jnp.minimum(root_ref[gi, 0], float(N - 1)).astype(jnp.int32)
            xr_ref[pl.ds(gi, 1), :] = x_ref[pl.ds(ri, 1), :]
        news = jnp.maximum(
            jnp.dot(xr_ref[...], lnw_ref[...],
                    preferred_element_type=jnp.float32) + lnb_ref[...], 0.0)
        z = jnp.concatenate([hg, news], axis=1)
        val = jnp.dot(z, l1w_ref[...],
                      preferred_element_type=jnp.float32) + l1b_ref[0]
        out_ref[...] = jax.nn.sigmoid(val)


def _tc_final(p0, p1, b, g, be, batf, x, lnW, lnb, l0W, l0b, l1W, l1b):
    return pl.pallas_call(
        _fin_body,
        grid=(3, NB),
        in_specs=[
            pl.BlockSpec((R, C2), lambda p, i: (i, 0)),
            pl.BlockSpec((R, C2), lambda p, i: (i, 0)),
            pl.BlockSpec((1, C), lambda p, i: (0, 0)),
            pl.BlockSpec((1, C), lambda p, i: (0, 0)),
            pl.BlockSpec((1, C), lambda p, i: (0, 0)),
            pl.BlockSpec((R, 1), lambda p, i: (i, 0)),
            pl.BlockSpec((N, D), lambda p, i: (0, 0)),
            pl.BlockSpec((D, C), lambda p, i: (0, 0)),
            pl.BlockSpec((1, C), lambda p, i: (0, 0)),
            pl.BlockSpec((2 * C, C), lambda p, i: (0, 0)),
            pl.BlockSpec((1, C), lambda p, i: (0, 0)),
            pl.BlockSpec((2 * C, 1), lambda p, i: (0, 0)),
            pl.BlockSpec(memory_space=pltpu.SMEM),
        ],
        out_specs=pl.BlockSpec((G, 1), lambda p, i: (0, 0)),
        out_shape=jax.ShapeDtypeStruct((G, 1), jnp.float32),
        scratch_shapes=[
            pltpu.VMEM((1, C), jnp.float32),
            pltpu.VMEM((1, C), jnp.float32),
            pltpu.VMEM((G, C), jnp.float32),
            pltpu.VMEM((G, C), jnp.float32),
            pltpu.VMEM((G, 1), jnp.float32),
            pltpu.VMEM((G, 1), jnp.float32),
            pltpu.VMEM((G, D), jnp.float32),
        ],
        compiler_params=pltpu.CompilerParams(
            dimension_semantics=("arbitrary", "arbitrary")),
    )(p0, p1, b, g, be, batf, x, lnW, lnb, l0W, l0b, l1W, l1b)


def _block_diag(a):
    m = a[:, :, None] * jnp.eye(H, dtype=a.dtype)[:, None, :]
    return jnp.pad(m.reshape(HC, H), ((0, 0), (0, AW - H)))


def _gat_layer(xh0, xh1, als, ald, srcp, dstp, z16, z64):
    ex, den = _sc_pass1(als, ald, srcp, dstp, z16)
    return _sc_pass2(xh0, xh1, ex, den[0], den[1], srcp, dstp, z64)


def kernel(x, edge_index, batch, W1, as1, ad1, b1, g1, be1,
           W2, as2, ad2, b2, g2, be2, W3, as3, ad3, b3, g3, be3,
           lnW, lnb, l0W, l0b, l1W, l1b):
    loop = jnp.arange(N, dtype=edge_index.dtype)
    pad = jnp.zeros((E_PAD - E_TOT,), edge_index.dtype)
    srcp = jnp.concatenate([edge_index[0], loop, pad])
    dstp = jnp.concatenate([edge_index[1], loop, pad])
    z16 = jnp.zeros((N, AW), jnp.float32)
    z64 = jnp.zeros((N, C2), jnp.float32)
    batf = batch.astype(jnp.float32).reshape(N, 1)
    perm = jnp.asarray(PERM)

    xh0, xh1, als, ald = _tc_prep(
        x, W1[:, perm], _block_diag(as1)[perm, :], _block_diag(ad1)[perm, :])
    p0, p1 = _gat_layer(xh0, xh1, als, ald, srcp, dstp, z16, z64)
    xh0, xh1, als, ald = _tc_mid(
        p0, p1, b1.reshape(1, C), g1.reshape(1, C), be1.reshape(1, C),
        W2[:, perm], _block_diag(as2)[perm, :], _block_diag(ad2)[perm, :])
    p0, p1 = _gat_layer(xh0, xh1, als, ald, srcp, dstp, z16, z64)
    xh0, xh1, als, ald = _tc_mid(
        p0, p1, b2.reshape(1, C), g2.reshape(1, C), be2.reshape(1, C),
        W3[:, perm], _block_diag(as3)[perm, :], _block_diag(ad3)[perm, :])
    p0, p1 = _gat_layer(xh0, xh1, als, ald, srcp, dstp, z16, z64)
    return _tc_final(p0, p1, b3.reshape(1, C), g3.reshape(1, C),
                     be3.reshape(1, C), batf, x,
                     lnW, lnb.reshape(1, C), l0W, l0b.reshape(1, C),
                     l1W, l1b)


# speedup vs baseline: 27.9817x; 1.0518x over previous
import functools
import jax
import jax.numpy as jnp
import numpy as np
from jax import lax
from jax.experimental import pallas as pl
from jax.experimental.pallas import tpu as pltpu
from jax.experimental.pallas import tpu_sc as plsc

N = 10000
D = 128
H = 8
C = 128
G = 64
HC = H * C
AW = 16

NC = 2
NS = 16
NW = NC * NS
L = 16
C2 = C // NC
HC2 = H * C2

R = 1000
NB = N // R

E_RAW = 160000
E_TOT = E_RAW + N
KPW = 5376
E_PAD = KPW * NW
B1 = 128
NBLK1 = KPW // B1
B2 = 64
K2 = E_PAD // NS
NBLK2 = K2 // B2
CH = 632
CHL = N - (NS - 1) * CH

PERM = np.array([h * C + half * C2 + j
                 for half in range(NC) for h in range(H) for j in range(C2)],
                dtype=np.int32)
def _prep_body(x_ref, w_ref, as_ref, ad_ref, xh0_ref, xh1_ref,
               als_ref, ald_ref):
    xh = jnp.dot(x_ref[...], w_ref[...], preferred_element_type=jnp.float32)
    xh0_ref[...] = xh[:, :HC2]
    xh1_ref[...] = xh[:, HC2:]
    als_ref[...] = jnp.dot(xh, as_ref[...], preferred_element_type=jnp.float32)
    ald_ref[...] = jnp.dot(xh, ad_ref[...], preferred_element_type=jnp.float32)


def _tc_prep(x, W, As, Ad):
    din = x.shape[1]
    return pl.pallas_call(
        _prep_body,
        grid=(NB,),
        in_specs=[
            pl.BlockSpec((R, din), lambda i: (i, 0)),
            pl.BlockSpec((din, HC), lambda i: (0, 0)),
            pl.BlockSpec((HC, AW), lambda i: (0, 0)),
            pl.BlockSpec((HC, AW), lambda i: (0, 0)),
        ],
        out_specs=[
            pl.BlockSpec((R, HC2), lambda i: (i, 0)),
            pl.BlockSpec((R, HC2), lambda i: (i, 0)),
            pl.BlockSpec((R, AW), lambda i: (i, 0)),
            pl.BlockSpec((R, AW), lambda i: (i, 0)),
        ],
        out_shape=[
            jax.ShapeDtypeStruct((N, HC2), jnp.float32),
            jax.ShapeDtypeStruct((N, HC2), jnp.float32),
            jax.ShapeDtypeStruct((N, AW), jnp.float32),
            jax.ShapeDtypeStruct((N, AW), jnp.float32),
        ],
    )(x, W, As, Ad)


def _mid_body(p0_ref, p1_ref, b_ref, g_ref, be_ref, w_ref, as_ref, ad_ref,
              xh0_ref, xh1_ref, als_ref, ald_ref, sum_ref, ssq_ref):
    ph = pl.program_id(0)
    i = pl.program_id(1)

    @pl.when(jnp.logical_and(ph == 0, i == 0))
    def _():
        sum_ref[...] = jnp.zeros_like(sum_ref)
        ssq_ref[...] = jnp.zeros_like(ssq_ref)

    t = jnp.maximum(
        jnp.concatenate([p0_ref[...], p1_ref[...]], axis=1) + b_ref[...], 0.0)

    @pl.when(ph == 0)
    def _():
        sum_ref[...] += jnp.sum(t, 0, keepdims=True)
        ssq_ref[...] += jnp.sum(t * t, 0, keepdims=True)

    @pl.when(ph == 1)
    def _():
        mu = sum_ref[...] * (1.0 / N)
        var = ssq_ref[...] * (1.0 / N) - mu * mu
        hn = (t - mu) * lax.rsqrt(var + 1e-5) * g_ref[...] + be_ref[...]
        xh = jnp.dot(hn, w_ref[...], preferred_element_type=jnp.float32)
        xh0_ref[...] = xh[:, :HC2]
        xh1_ref[...] = xh[:, HC2:]
        als_ref[...] = jnp.dot(xh, as_ref[...],
                               preferred_element_type=jnp.float32)
        ald_ref[...] = jnp.dot(xh, ad_ref[...],
                               preferred_element_type=jnp.float32)


def _tc_mid(p0, p1, b, g, be, W, As, Ad):
    return pl.pallas_call(
        _mid_body,
        grid=(2, NB),
        in_specs=[
            pl.BlockSpec((R, C2), lambda p, i: (i, 0)),
            pl.BlockSpec((R, C2), lambda p, i: (i, 0)),
            pl.BlockSpec((1, C), lambda p, i: (0, 0)),
            pl.BlockSpec((1, C), lambda p, i: (0, 0)),
            pl.BlockSpec((1, C), lambda p, i: (0, 0)),
            pl.BlockSpec((C, HC), lambda p, i: (0, 0)),
            pl.BlockSpec((HC, AW), lambda p, i: (0, 0)),
            pl.BlockSpec((HC, AW), lambda p, i: (0, 0)),
        ],
        out_specs=[
            pl.BlockSpec((R, HC2), lambda p, i: (i, 0)),
            pl.BlockSpec((R, HC2), lambda p, i: (i, 0)),
            pl.BlockSpec((R, AW), lambda p, i: (i, 0)),
            pl.BlockSpec((R, AW), lambda p, i: (i, 0)),
        ],
        out_shape=[
            jax.ShapeDtypeStruct((N, HC2), jnp.float32),
            jax.ShapeDtypeStruct((N, HC2), jnp.float32),
            jax.ShapeDtypeStruct((N, AW), jnp.float32),
            jax.ShapeDtypeStruct((N, AW), jnp.float32),
        ],
        scratch_shapes=[
            pltpu.VMEM((1, C), jnp.float32),
            pltpu.VMEM((1, C), jnp.float32),
        ],
        compiler_params=pltpu.CompilerParams(
            dimension_semantics=("arbitrary", "arbitrary")),
    )(p0, p1, b, g, be, W, As, Ad)


def _sc_pass1(als, ald, srcp, dstp, z16):
    mesh = plsc.VectorSubcoreMesh(core_axis_name="c", subcore_axis_name="s")

    @functools.partial(
        pl.kernel,
        out_type=(
            jax.ShapeDtypeStruct((E_PAD, AW), jnp.float32),
            jax.ShapeDtypeStruct((NC, N, AW), jnp.float32),
        ),
        mesh=mesh,
        scratch_types=(
            [pltpu.VMEM((B1,), jnp.int32)] * 4
            + [pltpu.VMEM((B1, AW), jnp.float32)] * 6
            + [pltpu.VMEM_SHARED((N, AW), jnp.float32)]
            + [pltpu.SemaphoreType.DMA] * 8
        ),
        compiler_params=pltpu.CompilerParams(use_tc_tiling_on_sc=False),
    )
    def k(als_h, ald_h, src_h, dst_h, z_h, ex_h, den_h,
          idx_s0, idx_s1, idx_d0, idx_d1, sbuf0, sbuf1, dbuf0, dbuf1,
          exbuf0, exbuf1, den_sh,
          sem_s0, sem_s1, sem_d0, sem_d1, sem_w0, sem_w1, sem_x0, sem_x1):
        c = lax.axis_index("c")
        s = lax.axis_index("s")
        wid = s * NC + c
        offs = pl.multiple_of(s * CH, 8)

        base_w = wid * KPW
        bufs = (
            (idx_s0, idx_d0, sbuf0, dbuf0, exbuf0,
             sem_s0, sem_d0, sem_w0, sem_x0),
            (idx_s1, idx_d1, sbuf1, dbuf1, exbuf1,
             sem_s1, sem_d1, sem_w1, sem_x1),
        )

        def fire(i, b):
            idx_s, idx_d, sbuf, dbuf, exbuf, sem_s, sem_d, sem_w, sem_x \
                = bufs[b]
            base = base_w + i * B1

            @pl.when(i >= 2)
            def _():
                pltpu.make_async_copy(
                    exbuf, den_sh.at[idx_d], sem_w).wait()
                pltpu.make_async_copy(
                    exbuf, ex_h.at[pl.ds(base, B1), :], sem_x).wait()

            pltpu.sync_copy(src_h.at[pl.ds(base, B1)], idx_s)
            pltpu.sync_copy(dst_h.at[pl.ds(base, B1)], idx_d)
            pltpu.async_copy(als_h.at[idx_s], sbuf, sem_s)
            pltpu.async_copy(ald_h.at[idx_d], dbuf, sem_d)

        def consume(i, b):
            idx_s, idx_d, sbuf, dbuf, exbuf, sem_s, sem_d, sem_w, sem_x \
                = bufs[b]
            base = base_w + i * B1
            pltpu.make_async_copy(als_h.at[idx_s], sbuf, sem_s).wait()
            pltpu.make_async_copy(ald_h.at[idx_d], dbuf, sem_d).wait()
            for j in range(B1):
                a = sbuf[j, :] + dbuf[j, :]
                a = jnp.maximum(a, 0.2 * a)
                ev = jnp.exp(a)
                ev = jnp.where(base + j < E_TOT, ev, 0.0)
                exbuf[j, :] = ev
            pltpu.async_copy(exbuf, den_sh.at[idx_d], sem_w, add=True)
            pltpu.async_copy(exbuf, ex_h.at[pl.ds(base, B1), :], sem_x)

        fire(0, 0)

        @pl.when(s < NS - 1)
        def _():
            pltpu.sync_copy(z_h.at[pl.ds(offs, CH), :],
                            den_sh.at[pl.ds(offs, CH), :])

        @pl.when(s == NS - 1)
        def _():
            pltpu.sync_copy(z_h.at[pl.ds(offs, CHL), :],
                            den_sh.at[pl.ds(offs, CHL), :])

        plsc.subcore_barrier()

        def superblk(si, carry):
            for b in range(2):
                i = 2 * si + b

                @pl.when(i + 1 < NBLK1)
                def _():
                    fire(i + 1, 1 - b)

                consume(i, b)
            return carry

        lax.fori_loop(0, NBLK1 // 2, superblk, 0)
        for b in range(2):
            idx_s, idx_d, sbuf, dbuf, exbuf, sem_s, sem_d, sem_w, sem_x \
                = bufs[b]
            pltpu.make_async_copy(exbuf, den_sh.at[idx_d], sem_w).wait()
            pltpu.make_async_copy(
                exbuf, ex_h.at[pl.ds(0, B1), :], sem_x).wait()
        plsc.subcore_barrier()

        @pl.when(s < NS - 1)
        def _():
            pltpu.sync_copy(den_sh.at[pl.ds(offs, CH), :],
                            den_h.at[c, pl.ds(offs, CH), :])

        @pl.when(s == NS - 1)
        def _():
            pltpu.sync_copy(den_sh.at[pl.ds(offs, CHL), :],
                            den_h.at[c, pl.ds(offs, CHL), :])

    return k(als, ald, srcp, dstp, z16)


def _sc_pass2(xh0, xh1, ex, d0, d1, srcp, dstp, z64):
    mesh = plsc.VectorSubcoreMesh(core_axis_name="c", subcore_axis_name="s")

    @functools.partial(
        pl.kernel,
        out_type=(
            jax.ShapeDtypeStruct((N, C2), jnp.float32),
            jax.ShapeDtypeStruct((N, C2), jnp.float32),
        ),
        mesh=mesh,
        scratch_types=(
            [pltpu.VMEM((B2,), jnp.int32)] * 4
            + [pltpu.VMEM((B2, HC2), jnp.float32)] * 2
            + [pltpu.VMEM((B2, AW), jnp.float32)] * 8
            + [
                pltpu.VMEM((B2, C2), jnp.float32),
                pltpu.VMEM_SHARED((N, C2), jnp.float32),
            ]
            + [pltpu.SemaphoreType.DMA] * 6
        ),
        compiler_params=pltpu.CompilerParams(use_tc_tiling_on_sc=False),
    )
    def k(xh0_h, xh1_h, ex_h, d0_h, d1_h, src_h, dst_h, z_h,
          out0_h, out1_h,
          idx_s0, idx_s1, idx_d0, idx_d1, rows0, rows1,
          exb0, exb1, d0b0, d0b1, d1b0, d1b1, wb0, wb1, msg, out_sh,
          sem_r0, sem_r1, sem_a0, sem_a1, sem_b0, sem_b1):
        c = lax.axis_index("c")
        s = lax.axis_index("s")
        offs = pl.multiple_of(s * CH, 8)

        base_w = s * K2
        bufs = (
            (idx_s0, idx_d0, rows0, exb0, d0b0, d1b0, wb0,
             sem_r0, sem_a0, sem_b0),
            (idx_s1, idx_d1, rows1, exb1, d0b1, d1b1, wb1,
             sem_r1, sem_a1, sem_b1),
        )

        def run_core(xh_h):
            def fire(i, b):
                idx_s, idx_d, rows, exb, d0b, d1b, wb, sem_r, sem_a, sem_b \
                    = bufs[b]
                base = base_w + i * B2
                pltpu.sync_copy(src_h.at[pl.ds(base, B2)], idx_s)
                pltpu.sync_copy(dst_h.at[pl.ds(base, B2)], idx_d)
                pltpu.async_copy(xh_h.at[idx_s], rows, sem_r)
                pltpu.async_copy(d0_h.at[idx_d], d0b, sem_a)
                pltpu.async_copy(d1_h.at[idx_d], d1b, sem_b)
                pltpu.sync_copy(ex_h.at[pl.ds(base, B2), :], exb)

            def consume(b):
                idx_s, idx_d, rows, exb, d0b, d1b, wb, sem_r, sem_a, sem_b \
                    = bufs[b]
                pltpu.make_async_copy(d0_h.at[idx_d], d0b, sem_a).wait()
                pltpu.make_async_copy(d1_h.at[idx_d], d1b, sem_b).wait()
                for j in range(B2):
                    wb[j, :] = exb[j, :] * (1.0 / H) / (
                        d0b[j, :] + d1b[j, :] + 1e-16)
                pltpu.make_async_copy(xh_h.at[idx_s], rows, sem_r).wait()

                @plsc.parallel_loop(0, B2, unroll=4)
                def _edge(j):
                    wv = wb[j, :]
                    acc = [jnp.zeros((L,), jnp.float32)
                           for _ in range(C2 // L)]
                    for h in range(H):
                        sv = jnp.full((L,), wv[h])
                        for kk in range(C2 // L):
                            acc[kk] = acc[kk] + sv * rows[
                                j, pl.ds(h * C2 + kk * L, L)]
                    for kk in range(C2 // L):
                        msg[j, pl.ds(kk * L, L)] = acc[kk]

                pltpu.sync_copy(msg, out_sh.at[idx_d], add=True)

            fire(0, 0)

            @pl.when(s < NS - 1)
            def _():
                pltpu.sync_copy(z_h.at[pl.ds(offs, CH), :],
                                out_sh.at[pl.ds(offs, CH), :])

            @pl.when(s == NS - 1)
            def _():
                pltpu.sync_copy(z_h.at[pl.ds(offs, CHL), :],
                                out_sh.at[pl.ds(offs, CHL), :])

            plsc.subcore_barrier()

            def superblk(si, carry):
                for b in range(2):
                    i = 2 * si + b

                    @pl.when(i + 1 < NBLK2)
                    def _():
                        fire(i + 1, 1 - b)

                    consume(b)
                return carry

            lax.fori_loop(0, NBLK2 // 2, superblk, 0)

        @pl.when(c == 0)
        def _():
            run_core(xh0_h)

        @pl.when(c == 1)
        def _():
            run_core(xh1_h)

        plsc.subcore_barrier()

        @pl.when(c == 0)
        def _():
            @pl.when(s < NS - 1)
            def _():
                pltpu.sync_copy(out_sh.at[pl.ds(offs, CH), :],
                                out0_h.at[pl.ds(offs, CH), :])

            @pl.when(s == NS - 1)
            def _():
                pltpu.sync_copy(out_sh.at[pl.ds(offs, CHL), :],
                                out0_h.at[pl.ds(offs, CHL), :])

        @pl.when(c == 1)
        def _():
            @pl.when(s < NS - 1)
            def _():
                pltpu.sync_copy(out_sh.at[pl.ds(offs, CH), :],
                                out1_h.at[pl.ds(offs, CH), :])

            @pl.when(s == NS - 1)
            def _():
                pltpu.sync_copy(out_sh.at[pl.ds(offs, CHL), :],
                                out1_h.at[pl.ds(offs, CHL), :])

    return k(xh0, xh1, ex, d0, d1, srcp, dstp, z64)


def _fin_body(p0_ref, p1_ref, b_ref, g_ref, be_ref, bat_ref, x_ref,
              lnw_ref, lnb_ref, l0w_ref, l0b_ref, l1w_ref, l1b_ref,
              out_ref,
              sum_ref, ssq_ref, gmax_ref, gsum_ref, cnt_ref, root_ref,
              xr_ref):
    ph = pl.program_id(0)
    i = pl.program_id(1)

    @pl.when(jnp.logical_and(ph == 0, i == 0))
    def _():
        sum_ref[...] = jnp.zeros_like(sum_ref)
        ssq_ref[...] = jnp.zeros_like(ssq_ref)
        gmax_ref[...] = jnp.full_like(gmax_ref, -jnp.inf)
        gsum_ref[...] = jnp.zeros_like(gsum_ref)
        cnt_ref[...] = jnp.zeros_like(cnt_ref)
        root_ref[...] = jnp.full_like(root_ref, jnp.inf)

    t = jnp.maximum(
        jnp.concatenate([p0_ref[...], p1_ref[...]], axis=1) + b_ref[...], 0.0)

    @pl.when(ph == 0)
    def _():
        sum_ref[...] += jnp.sum(t, 0, keepdims=True)
        ssq_ref[...] += jnp.sum(t * t, 0, keepdims=True)

    @pl.when(ph == 1)
    def _():
        mu = sum_ref[...] * (1.0 / N)
        var = ssq_ref[...] * (1.0 / N) - mu * mu
        h = (t - mu) * lax.rsqrt(var + 1e-5) * g_ref[...] + be_ref[...]
        bat = bat_ref[...]
        rowf = (i * R + lax.broadcasted_iota(jnp.int32, (R, 1), 0)
                ).astype(jnp.float32)
        bmin = jnp.min(bat).astype(jnp.int32)
        bmax = jnp.max(bat).astype(jnp.int32)

        def body(gi, carry):
            @pl.when(jnp.logical_and(gi >= bmin, gi <= bmax))
            def _():
                mask = bat == gi.astype(jnp.float32)
                hm = jnp.where(mask, h, -jnp.inf)
                gmax_ref[pl.ds(gi, 1), :] = jnp.maximum(
                    gmax_ref[pl.ds(gi, 1), :], jnp.max(hm, 0, keepdims=True))
                hs = jnp.where(mask, h, 0.0)
                gsum_ref[pl.ds(gi, 1), :] += jnp.sum(hs, 0, keepdims=True)
                cnt_ref[pl.ds(gi, 1), :] += jnp.sum(
                    mask.astype(jnp.float32), 0, keepdims=True)
                ridx = jnp.where(mask, rowf, jnp.inf)
                root_ref[pl.ds(gi, 1), :] = jnp.minimum(
                    root_ref[pl.ds(gi, 1), :], jnp.min(ridx, 0, keepdims=True))
            return carry

        lax.fori_loop(0, G, body, 0)

    @pl.when(jnp.logical_and(ph == 2, i == 0))
    def _():
        gmax = gmax_ref[...]
        gmax = jnp.where(jnp.isfinite(gmax), gmax, 0.0)
        gmean = gsum_ref[...] / jnp.maximum(cnt_ref[...], 1.0)
        hgin = jnp.concatenate([gmax, gmean], axis=1)
        hg = jnp.maximum(
            jnp.dot(hgin, l0w_ref[...], preferred_element_type=jnp.float32)
            + l0b_ref[...], 0.0)
        for gi in range(G):
            ri = jnp.minimum(root_ref[gi, 0], float(N - 1)).astype(jnp.int32)
            xr_ref[pl.ds(gi, 1), :] = x_ref[pl.ds(ri, 1), :]
        news = jnp.maximum(
            jnp.dot(xr_ref[...], lnw_ref[...],
                    preferred_element_type=jnp.float32) + lnb_ref[...], 0.0)
        z = jnp.concatenate([hg, news], axis=1)
        val = jnp.dot(z, l1w_ref[...],
                      preferred_element_type=jnp.float32) + l1b_ref[0]
        out_ref[...] = jax.nn.sigmoid(val)


def _tc_final(p0, p1, b, g, be, batf, x, lnW, lnb, l0W, l0b, l1W, l1b):
    return pl.pallas_call(
        _fin_body,
        grid=(3, NB),
        in_specs=[
            pl.BlockSpec((R, C2), lambda p, i: (i, 0)),
            pl.BlockSpec((R, C2), lambda p, i: (i, 0)),
            pl.BlockSpec((1, C), lambda p, i: (0, 0)),
            pl.BlockSpec((1, C), lambda p, i: (0, 0)),
            pl.BlockSpec((1, C), lambda p, i: (0, 0)),
            pl.BlockSpec((R, 1), lambda p, i: (i, 0)),
            pl.BlockSpec((N, D), lambda p, i: (0, 0)),
            pl.BlockSpec((D, C), lambda p, i: (0, 0)),
            pl.BlockSpec((1, C), lambda p, i: (0, 0)),
            pl.BlockSpec((2 * C, C), lambda p, i: (0, 0)),
            pl.BlockSpec((1, C), lambda p, i: (0, 0)),
            pl.BlockSpec((2 * C, 1), lambda p, i: (0, 0)),
            pl.BlockSpec(memory_space=pltpu.SMEM),
        ],
        out_specs=pl.BlockSpec((G, 1), lambda p, i: (0, 0)),
        out_shape=jax.ShapeDtypeStruct((G, 1), jnp.float32),
        scratch_shapes=[
            pltpu.VMEM((1, C), jnp.float32),
            pltpu.VMEM((1, C), jnp.float32),
            pltpu.VMEM((G, C), jnp.float32),
            pltpu.VMEM((G, C), jnp.float32),
            pltpu.VMEM((G, 1), jnp.float32),
            pltpu.VMEM((G, 1), jnp.float32),
            pltpu.VMEM((G, D), jnp.float32),
        ],
        compiler_params=pltpu.CompilerParams(
            dimension_semantics=("arbitrary", "arbitrary")),
    )(p0, p1, b, g, be, batf, x, lnW, lnb, l0W, l0b, l1W, l1b)


def _block_diag(a):
    m = a[:, :, None] * jnp.eye(H, dtype=a.dtype)[:, None, :]
    return jnp.pad(m.reshape(HC, H), ((0, 0), (0, AW - H)))


def _gat_layer(xh0, xh1, als, ald, srcp, dstp, z16, z64):
    ex, den = _sc_pass1(als, ald, srcp, dstp, z16)
    return _sc_pass2(xh0, xh1, ex, den[0], den[1], srcp, dstp, z64)


def kernel(x, edge_index, batch, W1, as1, ad1, b1, g1, be1,
           W2, as2, ad2, b2, g2, be2, W3, as3, ad3, b3, g3, be3,
           lnW, lnb, l0W, l0b, l1W, l1b):
    loop = jnp.arange(N, dtype=edge_index.dtype)
    pad = jnp.zeros((E_PAD - E_TOT,), edge_index.dtype)
    srcp = jnp.concatenate([edge_index[0], loop, pad])
    dstp = jnp.concatenate([edge_index[1], loop, pad])
    z16 = jnp.zeros((N, AW), jnp.float32)
    z64 = jnp.zeros((N, C2), jnp.float32)
    batf = batch.astype(jnp.float32).reshape(N, 1)
    perm = jnp.asarray(PERM)

    xh0, xh1, als, ald = _tc_prep(
        x, W1[:, perm], _block_diag(as1)[perm, :], _block_diag(ad1)[perm, :])
    p0, p1 = _gat_layer(xh0, xh1, als, ald, srcp, dstp, z16, z64)
    xh0, xh1, als, ald = _tc_mid(
        p0, p1, b1.reshape(1, C), g1.reshape(1, C), be1.reshape(1, C),
        W2[:, perm], _block_diag(as2)[perm, :], _block_diag(ad2)[perm, :])
    p0, p1 = _gat_layer(xh0, xh1, als, ald, srcp, dstp, z16, z64)
    xh0, xh1, als, ald = _tc_mid(
        p0, p1, b2.reshape(1, C), g2.reshape(1, C), be2.reshape(1, C),
        W3[:, perm], _block_diag(as3)[perm, :], _block_diag(ad3)[perm, :])
    p0, p1 = _gat_layer(xh0, xh1, als, ald, srcp, dstp, z16, z64)
    return _tc_final(p0, p1, b3.reshape(1, C), g3.reshape(1, C),
                     be3.reshape(1, C), batf, x,
                     lnW, lnb.reshape(1, C), l0W, l0b.reshape(1, C),
                     l1W, l1b)
